# baseline (speedup 1.0000x reference)
import functools

import jax
import jax.numpy as jnp
import numpy as np
from jax.experimental import pallas as pl
from jax.experimental.pallas import tpu as pltpu

D = 1024
H = 1024
O = 1024
N = 8
TOK = 2048
TEMP = float(np.e)
NA = 5


def _moe_body(x_ref, wg_ref, bgt_ref, w1_ref, b1_ref, w2_ref, b2_ref,
              out_ref, w_ref, xbf_ref):
    e = pl.program_id(0)

    @pl.when(e == 0)
    def _gate():
        x = x_ref[...]
        logt = jax.lax.dot_general(
            wg_ref[...], x, (((1,), (1,)), ((), ())),
            preferred_element_type=jnp.float32) + bgt_ref[...]
        s = logt * (1.0 / TEMP)
        m = jnp.max(s, axis=0, keepdims=True)
        q = jnp.exp(s - m)
        p = q / jnp.sum(q, axis=0, keepdims=True)
        iota = jax.lax.broadcasted_iota(jnp.int32, (N, TOK), 0)
        pmk = p
        mask = jnp.zeros_like(p)
        for _ in range(NA):
            cm = jnp.max(pmk, axis=0, keepdims=True)
            first = jnp.min(jnp.where(pmk == cm, iota, N), axis=0,
                            keepdims=True)
            sel = iota == first
            mask = jnp.where(sel, 1.0, mask)
            pmk = jnp.where(sel, -1.0, pmk)
        wm = p * mask
        wt = wm / (jnp.sum(wm, axis=0, keepdims=True) + 1e-8)
        w_ref[...] = jnp.transpose(wt)
        xbf_ref[...] = x.astype(jnp.bfloat16)

    h1 = jax.lax.dot_general(
        xbf_ref[...], w1_ref[0].astype(jnp.bfloat16),
        (((1,), (1,)), ((), ())), preferred_element_type=jnp.float32)
    h1 = jnp.maximum(h1 + b1_ref[0], 0.0)
    eo = jax.lax.dot_general(
        h1.astype(jnp.bfloat16), w2_ref[0].astype(jnp.bfloat16),
        (((1,), (1,)), ((), ())), preferred_element_type=jnp.float32)
    eo = eo + b2_ref[0]
    iota = jax.lax.broadcasted_iota(jnp.int32, (TOK, N), 1)
    wcol = jnp.sum(jnp.where(iota == e, w_ref[...], 0.0), axis=1,
                   keepdims=True)
    prev = jnp.where(e == 0, 0.0, out_ref[...])
    out_ref[...] = prev + wcol * eo


@jax.jit
def kernel(x, Wg, bg, W1, b1, W2, b2):
    out, w = pl.pallas_call(
        _moe_body,
        grid=(N,),
        in_specs=[
            pl.BlockSpec((TOK, D), lambda e: (0, 0)),
            pl.BlockSpec((N, D), lambda e: (0, 0)),
            pl.BlockSpec((N, 1), lambda e: (0, 0)),
            pl.BlockSpec((1, H, D), lambda e: (e, 0, 0)),
            pl.BlockSpec((1, 1, H), lambda e: (e, 0, 0)),
            pl.BlockSpec((1, O, H), lambda e: (e, 0, 0)),
            pl.BlockSpec((1, 1, O), lambda e: (e, 0, 0)),
        ],
        out_specs=[
            pl.BlockSpec((TOK, O), lambda e: (0, 0)),
            pl.BlockSpec((TOK, N), lambda e: (0, 0)),
        ],
        out_shape=[
            jax.ShapeDtypeStruct((TOK, O), jnp.float32),
            jax.ShapeDtypeStruct((TOK, N), jnp.float32),
        ],
        scratch_shapes=[pltpu.VMEM((TOK, D), jnp.bfloat16)],
        compiler_params=pltpu.CompilerParams(
            dimension_semantics=("arbitrary",),
            vmem_limit_bytes=120 * 1024 * 1024),
    )(x, Wg, bg.reshape(N, 1), W1, b1.reshape(N, 1, H), W2,
      b2.reshape(N, 1, O))
    return (out, w)
